# baseline (speedup 1.0000x reference)
import functools

import jax
import jax.numpy as jnp
from jax.experimental import pallas as pl
from jax.experimental.pallas import tpu as pltpu

_NE = 64
_EPS = 1e-6
_TILE = 4096
_KSPLIT = 2


def _body(x_ref, w_ref, b_ref, o_ref, lt_s, v_s, e_s, dacc, r_s, *,
          num_tiles, capacity):
    p = pl.program_id(0)
    i = pl.program_id(1)
    k = pl.program_id(2)

    @pl.when(p == 0)
    def _phase0():
        part = jax.lax.dot_general(
            w_ref[...], x_ref[...], (((1,), (1,)), ((), ())),
            preferred_element_type=jnp.float32)

        @pl.when(k == 0)
        def _():
            lt_s[...] = part + b_ref[...]

        @pl.when(k == 1)
        def _():
            lt = lt_s[...] + part
            m = jnp.max(lt, axis=0, keepdims=True)
            s = jnp.sum(jnp.exp(lt - m), axis=0, keepdims=True)
            v = 1.0 / s
            iota = jax.lax.broadcasted_iota(jnp.int32, (_NE, _TILE), 0)
            e = jnp.min(jnp.where(lt == m, iota, _NE), axis=0, keepdims=True)
            v_s[i, 0, :] = v[0]
            e_s[i, 0, :] = e[0]
            contrib = jnp.where(iota == e, v, 0.0)

            @pl.when(i == 0)
            def _():
                dacc[...] = jnp.zeros_like(dacc)

            dacc[...] += contrib

            @pl.when(i == num_tiles - 1)
            def _():
                denom = jnp.sum(dacc[...], axis=1) + _EPS
                r_s[...] = (capacity / denom)[:, None]

    @pl.when((p == 1) & (k == 1))
    def _phase1():
        recip = r_s[...]
        v = v_s[i]
        e = e_s[i]
        iota = jax.lax.broadcasted_iota(jnp.int32, (_NE, _TILE), 0)
        out_t = jnp.where(iota == e, v * recip, 0.0)
        o_ref[...] = out_t.T


def kernel(x, w_gate, b_gate):
    n, dim = x.shape
    ne = w_gate.shape[0]
    capacity = float(n)
    num_tiles = n // _TILE
    kdim = dim // _KSPLIT
    b2 = b_gate.reshape(ne, 1)
    last = num_tiles - 1

    out = pl.pallas_call(
        functools.partial(_body, num_tiles=num_tiles, capacity=capacity),
        grid=(2, num_tiles, _KSPLIT),
        in_specs=[
            pl.BlockSpec(
                (_TILE, kdim),
                lambda p, i, k: ((1 - p) * i + p * last, (1 - p) * k + p)),
            pl.BlockSpec((ne, kdim), lambda p, i, k: (0, (1 - p) * k + p)),
            pl.BlockSpec((ne, 1), lambda p, i, k: (0, 0)),
        ],
        out_specs=pl.BlockSpec((_TILE, ne), lambda p, i, k: (p * i, 0)),
        out_shape=jax.ShapeDtypeStruct((n, ne), jnp.float32),
        scratch_shapes=[
            pltpu.VMEM((_NE, _TILE), jnp.float32),
            pltpu.VMEM((n // _TILE, 1, _TILE), jnp.float32),
            pltpu.VMEM((n // _TILE, 1, _TILE), jnp.int32),
            pltpu.VMEM((_NE, _TILE), jnp.float32),
            pltpu.VMEM((_NE, 1), jnp.float32),
        ],
        compiler_params=pltpu.CompilerParams(
            dimension_semantics=("arbitrary", "arbitrary", "arbitrary")),
    )(x, w_gate, b2)
    return out


# speedup vs baseline: 1.2760x; 1.2760x over previous
import functools

import jax
import jax.numpy as jnp
from jax.experimental import pallas as pl
from jax.experimental.pallas import tpu as pltpu

_NE = 64
_EPS = 1e-6
_TILE = 4096


def _body(x_ref, w_ref, b_ref, o_ref, v_s, e_s, dacc, r_s, *,
          num_tiles, capacity):
    p = pl.program_id(0)
    i = pl.program_id(1)

    @pl.when(p == 0)
    def _phase0():
        lt = jax.lax.dot_general(
            w_ref[...], x_ref[...], (((1,), (1,)), ((), ())),
            preferred_element_type=jnp.float32)
        lt = lt + b_ref[...]
        m = jnp.max(lt, axis=0, keepdims=True)
        ones_row = jnp.ones((1, _NE), jnp.float32)
        s = jax.lax.dot_general(
            ones_row, jnp.exp(lt - m), (((1,), (0,)), ((), ())),
            preferred_element_type=jnp.float32)
        v = 1.0 / s
        iota = jax.lax.broadcasted_iota(jnp.int32, (_NE, _TILE), 0)
        e = jnp.min(jnp.where(lt == m, iota, _NE), axis=0, keepdims=True)
        v_s[i, 0, :] = v[0]
        e_s[i, 0, :] = e[0]
        contrib = jnp.where(iota == e, v, 0.0)
        ones_col = jnp.ones((_TILE, 1), jnp.float32)
        dpart = jax.lax.dot_general(
            contrib, ones_col, (((1,), (0,)), ((), ())),
            preferred_element_type=jnp.float32)

        @pl.when(i == 0)
        def _():
            dacc[...] = jnp.zeros_like(dacc)

        dacc[...] += dpart

        @pl.when(i == num_tiles - 1)
        def _():
            r_s[...] = capacity / (dacc[...] + _EPS)

    @pl.when(p == 1)
    def _phase1():
        recip = r_s[...]
        v = v_s[i]
        e = e_s[i]
        iota = jax.lax.broadcasted_iota(jnp.int32, (_NE, _TILE), 0)
        out_t = jnp.where(iota == e, v * recip, 0.0)
        o_ref[...] = out_t.T


def kernel(x, w_gate, b_gate):
    n, dim = x.shape
    ne = w_gate.shape[0]
    capacity = float(n)
    num_tiles = n // _TILE
    b2 = b_gate.reshape(ne, 1)
    last = num_tiles - 1

    out = pl.pallas_call(
        functools.partial(_body, num_tiles=num_tiles, capacity=capacity),
        grid=(2, num_tiles),
        in_specs=[
            pl.BlockSpec((_TILE, dim), lambda p, i: ((1 - p) * i + p * last,
                                                     0)),
            pl.BlockSpec((ne, dim), lambda p, i: (0, 0)),
            pl.BlockSpec((ne, 1), lambda p, i: (0, 0)),
        ],
        out_specs=pl.BlockSpec((_TILE, ne), lambda p, i: (p * i, 0)),
        out_shape=jax.ShapeDtypeStruct((n, ne), jnp.float32),
        scratch_shapes=[
            pltpu.VMEM((n // _TILE, 1, _TILE), jnp.float32),
            pltpu.VMEM((n // _TILE, 1, _TILE), jnp.int32),
            pltpu.VMEM((_NE, 1), jnp.float32),
            pltpu.VMEM((_NE, 1), jnp.float32),
        ],
        compiler_params=pltpu.CompilerParams(
            dimension_semantics=("arbitrary", "arbitrary")),
    )(x, w_gate, b2)
    return out
